# baseline (speedup 1.0000x reference)
import functools

import jax
import jax.numpy as jnp
from jax import lax
from jax.experimental import pallas as pl
from jax.experimental.pallas import tpu as pltpu
from jax.experimental.pallas import tpu_sc as plsc

D_MODEL = 512
HALF = D_MODEL // 2
MAX_SIZE = 512
N = 100000
CHUNK = 40
NUM_CHUNKS = N // CHUNK
NUM_WORKERS = 32
BASE_ITERS = NUM_CHUNKS // NUM_WORKERS
EXTRA = NUM_CHUNKS - BASE_ITERS * NUM_WORKERS
MAX_ITERS = BASE_ITERS + 1
DEPTH = 2
IDX_PAD = MAX_ITERS * CHUNK


def _gather_body(xs_hbm, ys_hbm, pe_hbm, out_hbm, x_v, y_v, rows_v, sem0, sem1):
    sems = (sem0, sem1)
    wid = lax.axis_index("s") * 2 + lax.axis_index("c")
    start = wid * BASE_ITERS + lax.min(wid, EXTRA)
    count = BASE_ITERS + jnp.where(wid < EXTRA, 1, 0)
    last = count - 1
    pre = pl.multiple_of(start * CHUNK, 8)
    pltpu.sync_copy(xs_hbm.at[pl.ds(pre, IDX_PAD)], x_v)
    pltpu.sync_copy(ys_hbm.at[pl.ds(pre, IDX_PAD)], y_v)

    def issue(slot, b):
        off = pl.multiple_of(lax.min(slot, last) * CHUNK, 8)
        pltpu.async_copy(
            pe_hbm.at[x_v.at[pl.ds(off, CHUNK)]],
            rows_v.at[b, slice(None), pl.ds(0, HALF)],
            sems[b],
        )
        pltpu.async_copy(
            pe_hbm.at[y_v.at[pl.ds(off, CHUNK)]],
            rows_v.at[b, slice(None), pl.ds(HALF, HALF)],
            sems[b],
        )

    def wait(b):
        pltpu.make_async_copy(
            pe_hbm.at[x_v.at[pl.ds(0, CHUNK)]],
            rows_v.at[b, slice(None), pl.ds(0, HALF)],
            sems[b],
        ).wait()
        pltpu.make_async_copy(
            pe_hbm.at[y_v.at[pl.ds(0, CHUNK)]],
            rows_v.at[b, slice(None), pl.ds(HALF, HALF)],
            sems[b],
        ).wait()

    def store(slot, b):
        base = pl.multiple_of((start + lax.min(slot, last)) * CHUNK, 8)
        pltpu.sync_copy(rows_v.at[b], out_hbm.at[pl.ds(base, CHUNK)])

    for s in range(DEPTH - 1):
        issue(s, s)
    steps = (count + DEPTH - 1) >> 1

    def body(j, carry):
        for b in range(DEPTH):
            slot = DEPTH * j + b
            issue(slot + DEPTH - 1, (b + DEPTH - 1) % DEPTH)
            wait(b)
            store(slot, b)
        return carry

    lax.fori_loop(0, steps, body, 0)
    for b in range(DEPTH - 1):
        wait(b)


@jax.jit
def _pe_lookup(xs, ys, pe):
    mesh = plsc.VectorSubcoreMesh(core_axis_name="c", subcore_axis_name="s")
    f = functools.partial(
        pl.kernel,
        mesh=mesh,
        out_type=jax.ShapeDtypeStruct((N, D_MODEL), jnp.float32),
        scratch_types=[
            pltpu.VMEM((IDX_PAD,), jnp.int32),
            pltpu.VMEM((IDX_PAD,), jnp.int32),
            pltpu.VMEM((DEPTH, CHUNK, D_MODEL), jnp.float32),
            pltpu.SemaphoreType.DMA,
            pltpu.SemaphoreType.DMA,
        ],
    )(_gather_body)
    return f(xs, ys, pe)


REPL = 32


def kernel(coords, pe):
    cids = jnp.clip(coords.astype(jnp.int32), 0, MAX_SIZE - 1)
    c = jnp.arange(N, dtype=jnp.int32) // CHUNK
    offx = ((2 * c) % REPL) * MAX_SIZE
    offy = ((2 * c + 1) % REPL) * MAX_SIZE
    pad = jnp.zeros((CHUNK,), jnp.int32)
    xs = jnp.concatenate([cids[:, 0] + offx, pad])
    ys = jnp.concatenate([cids[:, 1] + offy, pad])
    pe_rep = jnp.concatenate([pe] * REPL)
    return _pe_lookup(xs, ys, pe_rep)


# speedup vs baseline: 1.1032x; 1.1032x over previous
import functools

import jax
import jax.numpy as jnp
from jax import lax
from jax.experimental import pallas as pl
from jax.experimental.pallas import tpu as pltpu
from jax.experimental.pallas import tpu_sc as plsc

D_MODEL = 512
HALF = D_MODEL // 2
MAX_SIZE = 512
N = 100000
CHUNK = 40
NUM_CHUNKS = N // CHUNK
NUM_WORKERS = 32
BASE_ITERS = NUM_CHUNKS // NUM_WORKERS
EXTRA = NUM_CHUNKS - BASE_ITERS * NUM_WORKERS
MAX_ITERS = BASE_ITERS + 1
DEPTH = 2
IDX_PAD = MAX_ITERS * CHUNK


def _gather_body(xs_hbm, ys_hbm, pe_hbm, out_hbm, x_v, y_v, rows_v, sem0, sem1):
    sems = (sem0, sem1)
    wid = lax.axis_index("s") * 2 + lax.axis_index("c")
    start = wid * BASE_ITERS + lax.min(wid, EXTRA)
    count = BASE_ITERS + jnp.where(wid < EXTRA, 1, 0)
    last = count - 1
    pre = pl.multiple_of(start * CHUNK, 8)
    pltpu.sync_copy(xs_hbm.at[pl.ds(pre, IDX_PAD)], x_v)
    pltpu.sync_copy(ys_hbm.at[pl.ds(pre, IDX_PAD)], y_v)

    def issue(slot, b):
        off = pl.multiple_of(lax.min(slot, last) * CHUNK, 8)
        pltpu.async_copy(
            pe_hbm.at[x_v.at[pl.ds(off, CHUNK)]],
            rows_v.at[b, slice(None), pl.ds(0, HALF)],
            sems[b],
        )
        pltpu.async_copy(
            pe_hbm.at[y_v.at[pl.ds(off, CHUNK)]],
            rows_v.at[b, slice(None), pl.ds(HALF, HALF)],
            sems[b],
        )

    def wait(b):
        pltpu.make_async_copy(
            pe_hbm.at[x_v.at[pl.ds(0, CHUNK)]],
            rows_v.at[b, slice(None), pl.ds(0, HALF)],
            sems[b],
        ).wait()
        pltpu.make_async_copy(
            pe_hbm.at[y_v.at[pl.ds(0, CHUNK)]],
            rows_v.at[b, slice(None), pl.ds(HALF, HALF)],
            sems[b],
        ).wait()

    def store(slot, b):
        base = pl.multiple_of((start + lax.min(slot, last)) * CHUNK, 8)
        pltpu.sync_copy(rows_v.at[b], out_hbm.at[pl.ds(base, CHUNK)])

    for s in range(DEPTH - 1):
        issue(s, s)
    steps = (count + DEPTH - 1) >> 1

    def body(j, carry):
        for b in range(DEPTH):
            slot = DEPTH * j + b
            issue(slot + DEPTH - 1, (b + DEPTH - 1) % DEPTH)
            wait(b)
            store(slot, b)
        return carry

    lax.fori_loop(0, steps, body, 0)
    for b in range(DEPTH - 1):
        wait(b)


@jax.jit
def _pe_lookup(xs, ys, pe):
    mesh = plsc.VectorSubcoreMesh(core_axis_name="c", subcore_axis_name="s")
    f = functools.partial(
        pl.kernel,
        mesh=mesh,
        out_type=jax.ShapeDtypeStruct((N, D_MODEL), jnp.float32),
        scratch_types=[
            pltpu.VMEM((IDX_PAD,), jnp.int32),
            pltpu.VMEM((IDX_PAD,), jnp.int32),
            pltpu.VMEM((DEPTH, CHUNK, D_MODEL), jnp.float32),
            pltpu.SemaphoreType.DMA,
            pltpu.SemaphoreType.DMA,
        ],
    )(_gather_body)
    return f(xs, ys, pe)


REPL = 16


def kernel(coords, pe):
    cids = jnp.clip(coords.astype(jnp.int32), 0, MAX_SIZE - 1)
    c = jnp.arange(N, dtype=jnp.int32) // CHUNK
    offx = ((2 * c) % REPL) * MAX_SIZE
    offy = ((2 * c + 1) % REPL) * MAX_SIZE
    pad = jnp.zeros((CHUNK,), jnp.int32)
    xs = jnp.concatenate([cids[:, 0] + offx, pad])
    ys = jnp.concatenate([cids[:, 1] + offy, pad])
    pe_rep = jnp.concatenate([pe] * REPL)
    return _pe_lookup(xs, ys, pe_rep)


# speedup vs baseline: 1.1619x; 1.0532x over previous
import functools

import jax
import jax.numpy as jnp
from jax import lax
from jax.experimental import pallas as pl
from jax.experimental.pallas import tpu as pltpu
from jax.experimental.pallas import tpu_sc as plsc

D_MODEL = 512
HALF = D_MODEL // 2
MAX_SIZE = 512
N = 100000
CHUNK = 40
NUM_CHUNKS = N // CHUNK
NUM_WORKERS = 32
BASE_ITERS = NUM_CHUNKS // NUM_WORKERS
EXTRA = NUM_CHUNKS - BASE_ITERS * NUM_WORKERS
MAX_ITERS = BASE_ITERS + 1
DEPTH = 2
IDX_PAD = MAX_ITERS * CHUNK


def _gather_body(xs_hbm, ys_hbm, pe_hbm, out_hbm, x_v, y_v, rows_v, sem0, sem1):
    sems = (sem0, sem1)
    wid = lax.axis_index("s") * 2 + lax.axis_index("c")
    start = wid * BASE_ITERS + lax.min(wid, EXTRA)
    count = BASE_ITERS + jnp.where(wid < EXTRA, 1, 0)
    last = count - 1
    pre = pl.multiple_of(start * CHUNK, 8)
    pltpu.sync_copy(xs_hbm.at[pl.ds(pre, IDX_PAD)], x_v)
    pltpu.sync_copy(ys_hbm.at[pl.ds(pre, IDX_PAD)], y_v)

    def issue(slot, b):
        off = pl.multiple_of(lax.min(slot, last) * CHUNK, 8)
        pltpu.async_copy(
            pe_hbm.at[x_v.at[pl.ds(off, CHUNK)]],
            rows_v.at[b, slice(None), pl.ds(0, HALF)],
            sems[b],
        )
        pltpu.async_copy(
            pe_hbm.at[y_v.at[pl.ds(off, CHUNK)]],
            rows_v.at[b, slice(None), pl.ds(HALF, HALF)],
            sems[b],
        )

    def wait(b):
        pltpu.make_async_copy(
            pe_hbm.at[x_v.at[pl.ds(0, CHUNK)]],
            rows_v.at[b, slice(None), pl.ds(0, HALF)],
            sems[b],
        ).wait()
        pltpu.make_async_copy(
            pe_hbm.at[y_v.at[pl.ds(0, CHUNK)]],
            rows_v.at[b, slice(None), pl.ds(HALF, HALF)],
            sems[b],
        ).wait()

    def store(slot, b):
        base = pl.multiple_of((start + lax.min(slot, last)) * CHUNK, 8)
        pltpu.sync_copy(rows_v.at[b], out_hbm.at[pl.ds(base, CHUNK)])

    for s in range(DEPTH - 1):
        issue(s, s)
    steps = (count + DEPTH - 1) >> 1

    def body(j, carry):
        for b in range(DEPTH):
            slot = DEPTH * j + b
            issue(slot + DEPTH - 1, (b + DEPTH - 1) % DEPTH)
            wait(b)
            store(slot, b)
        return carry

    lax.fori_loop(0, steps, body, 0)
    for b in range(DEPTH - 1):
        wait(b)


@jax.jit
def _pe_lookup(xs, ys, pe):
    mesh = plsc.VectorSubcoreMesh(core_axis_name="c", subcore_axis_name="s")
    f = functools.partial(
        pl.kernel,
        mesh=mesh,
        out_type=jax.ShapeDtypeStruct((N, D_MODEL), jnp.float32),
        scratch_types=[
            pltpu.VMEM((IDX_PAD,), jnp.int32),
            pltpu.VMEM((IDX_PAD,), jnp.int32),
            pltpu.VMEM((DEPTH, CHUNK, D_MODEL), jnp.float32),
            pltpu.SemaphoreType.DMA,
            pltpu.SemaphoreType.DMA,
        ],
    )(_gather_body)
    return f(xs, ys, pe)


REPL = 8


def kernel(coords, pe):
    cids = jnp.clip(coords.astype(jnp.int32), 0, MAX_SIZE - 1)
    c = jnp.arange(N, dtype=jnp.int32)
    offx = ((2 * c) % REPL) * MAX_SIZE
    offy = ((2 * c + 1) % REPL) * MAX_SIZE
    pad = jnp.zeros((CHUNK,), jnp.int32)
    xs = jnp.concatenate([cids[:, 0] + offx, pad])
    ys = jnp.concatenate([cids[:, 1] + offy, pad])
    pe_rep = jnp.concatenate([pe] * REPL)
    return _pe_lookup(xs, ys, pe_rep)


# speedup vs baseline: 1.1733x; 1.0098x over previous
import functools

import jax
import jax.numpy as jnp
from jax import lax
from jax.experimental import pallas as pl
from jax.experimental.pallas import tpu as pltpu
from jax.experimental.pallas import tpu_sc as plsc

D_MODEL = 512
HALF = D_MODEL // 2
MAX_SIZE = 512
N = 100000
CHUNK = 40
NUM_CHUNKS = N // CHUNK
NUM_WORKERS = 32
BASE_ITERS = NUM_CHUNKS // NUM_WORKERS
EXTRA = NUM_CHUNKS - BASE_ITERS * NUM_WORKERS
MAX_ITERS = BASE_ITERS + 1
DEPTH = 2
IDX_PAD = MAX_ITERS * CHUNK


def _gather_body(xs_hbm, ys_hbm, pe_hbm, out_hbm, x_v, y_v, rows_v, sem0, sem1):
    sems = (sem0, sem1)
    wid = lax.axis_index("s") * 2 + lax.axis_index("c")
    start = wid * BASE_ITERS + lax.min(wid, EXTRA)
    count = BASE_ITERS + jnp.where(wid < EXTRA, 1, 0)
    last = count - 1
    pre = pl.multiple_of(lax.min(start * CHUNK, N - IDX_PAD), 8)
    delta = start * CHUNK - pre
    pltpu.sync_copy(xs_hbm.at[pl.ds(pre, IDX_PAD)], x_v)
    pltpu.sync_copy(ys_hbm.at[pl.ds(pre, IDX_PAD)], y_v)

    def issue(slot, b):
        off = pl.multiple_of(lax.min(slot, last) * CHUNK + delta, 8)
        pltpu.async_copy(
            pe_hbm.at[x_v.at[pl.ds(off, CHUNK)]],
            rows_v.at[b, slice(None), pl.ds(0, HALF)],
            sems[b],
        )
        pltpu.async_copy(
            pe_hbm.at[y_v.at[pl.ds(off, CHUNK)]],
            rows_v.at[b, slice(None), pl.ds(HALF, HALF)],
            sems[b],
        )

    def wait(b):
        pltpu.make_async_copy(
            pe_hbm.at[x_v.at[pl.ds(0, CHUNK)]],
            rows_v.at[b, slice(None), pl.ds(0, HALF)],
            sems[b],
        ).wait()
        pltpu.make_async_copy(
            pe_hbm.at[y_v.at[pl.ds(0, CHUNK)]],
            rows_v.at[b, slice(None), pl.ds(HALF, HALF)],
            sems[b],
        ).wait()

    def store(slot, b):
        base = pl.multiple_of((start + lax.min(slot, last)) * CHUNK, 8)
        pltpu.sync_copy(rows_v.at[b], out_hbm.at[pl.ds(base, CHUNK)])

    for s in range(DEPTH - 1):
        issue(s, s)
    steps = (count + DEPTH - 1) >> 1

    def body(j, carry):
        for b in range(DEPTH):
            slot = DEPTH * j + b
            issue(slot + DEPTH - 1, (b + DEPTH - 1) % DEPTH)
            wait(b)
            store(slot, b)
        return carry

    lax.fori_loop(0, steps, body, 0)
    for b in range(DEPTH - 1):
        wait(b)


@jax.jit
def _pe_lookup(xs, ys, pe):
    mesh = plsc.VectorSubcoreMesh(core_axis_name="c", subcore_axis_name="s")
    f = functools.partial(
        pl.kernel,
        mesh=mesh,
        out_type=jax.ShapeDtypeStruct((N, D_MODEL), jnp.float32),
        scratch_types=[
            pltpu.VMEM((IDX_PAD,), jnp.int32),
            pltpu.VMEM((IDX_PAD,), jnp.int32),
            pltpu.VMEM((DEPTH, CHUNK, D_MODEL), jnp.float32),
            pltpu.SemaphoreType.DMA,
            pltpu.SemaphoreType.DMA,
        ],
    )(_gather_body)
    return f(xs, ys, pe)


REPL = 8


def kernel(coords, pe):
    cids = jnp.clip(coords.astype(jnp.int32), 0, MAX_SIZE - 1)
    c = jnp.arange(N, dtype=jnp.int32)
    offx = ((2 * c) % REPL) * MAX_SIZE
    offy = ((2 * c + 1) % REPL) * MAX_SIZE
    xs = cids[:, 0] + offx
    ys = cids[:, 1] + offy
    pe_rep = jnp.concatenate([pe] * REPL)
    return _pe_lookup(xs, ys, pe_rep)


# speedup vs baseline: 1.1766x; 1.0028x over previous
import functools

import jax
import jax.numpy as jnp
from jax import lax
from jax.experimental import pallas as pl
from jax.experimental.pallas import tpu as pltpu
from jax.experimental.pallas import tpu_sc as plsc

D_MODEL = 512
HALF = D_MODEL // 2
MAX_SIZE = 512
N = 100000
CHUNK = 40
NUM_CHUNKS = N // CHUNK
NUM_WORKERS = 32
BASE_ITERS = NUM_CHUNKS // NUM_WORKERS
EXTRA = NUM_CHUNKS - BASE_ITERS * NUM_WORKERS
MAX_ITERS = BASE_ITERS + 1
DEPTH = 2
IDX_PAD = MAX_ITERS * CHUNK


def _gather_body(xs_hbm, ys_hbm, pe_hbm, out_hbm, x_v, y_v, rows_v, sem0, sem1):
    sems = (sem0, sem1)
    wid = lax.axis_index("s") * 2 + lax.axis_index("c")
    start = wid * BASE_ITERS + lax.min(wid, EXTRA)
    count = BASE_ITERS + jnp.where(wid < EXTRA, 1, 0)
    last = count - 1
    pre = pl.multiple_of(lax.min(start * CHUNK, N - IDX_PAD), 8)
    delta = start * CHUNK - pre
    pltpu.sync_copy(xs_hbm.at[pl.ds(pre, IDX_PAD)], x_v)
    pltpu.sync_copy(ys_hbm.at[pl.ds(pre, IDX_PAD)], y_v)

    def issue(slot, b):
        off = pl.multiple_of(lax.min(slot, last) * CHUNK + delta, 8)
        pltpu.async_copy(
            pe_hbm.at[x_v.at[pl.ds(off, CHUNK)]],
            rows_v.at[b, slice(None), pl.ds(0, HALF)],
            sems[b],
        )
        pltpu.async_copy(
            pe_hbm.at[y_v.at[pl.ds(off, CHUNK)]],
            rows_v.at[b, slice(None), pl.ds(HALF, HALF)],
            sems[b],
        )

    def wait(b):
        pltpu.make_async_copy(
            pe_hbm.at[x_v.at[pl.ds(0, CHUNK)]],
            rows_v.at[b, slice(None), pl.ds(0, HALF)],
            sems[b],
        ).wait()
        pltpu.make_async_copy(
            pe_hbm.at[y_v.at[pl.ds(0, CHUNK)]],
            rows_v.at[b, slice(None), pl.ds(HALF, HALF)],
            sems[b],
        ).wait()

    def store(slot, b):
        base = pl.multiple_of((start + lax.min(slot, last)) * CHUNK, 8)
        pltpu.sync_copy(rows_v.at[b], out_hbm.at[pl.ds(base, CHUNK)])

    for s in range(DEPTH - 1):
        issue(s, s)
    steps = (count + DEPTH - 1) >> 1

    def body(j, carry):
        for b in range(DEPTH):
            slot = DEPTH * j + b
            issue(slot + DEPTH - 1, (b + DEPTH - 1) % DEPTH)
            wait(b)
            store(slot, b)
        return carry

    lax.fori_loop(0, steps, body, 0)
    for b in range(DEPTH - 1):
        wait(b)


@jax.jit
def _pe_lookup(xs, ys, pe):
    mesh = plsc.VectorSubcoreMesh(core_axis_name="c", subcore_axis_name="s")
    f = functools.partial(
        pl.kernel,
        mesh=mesh,
        out_type=jax.ShapeDtypeStruct((N, D_MODEL), jnp.float32),
        scratch_types=[
            pltpu.VMEM((IDX_PAD,), jnp.int32),
            pltpu.VMEM((IDX_PAD,), jnp.int32),
            pltpu.VMEM((DEPTH, CHUNK, D_MODEL), jnp.float32),
            pltpu.SemaphoreType.DMA,
            pltpu.SemaphoreType.DMA,
        ],
    )(_gather_body)
    return f(xs, ys, pe)


REPL = 8


def kernel(coords, pe):
    cids = jnp.clip(coords.astype(jnp.int32), 0, MAX_SIZE - 1)
    c = jnp.arange(N, dtype=jnp.int32)
    offx = (c % REPL) * MAX_SIZE
    offy = ((c + REPL // 2) % REPL) * MAX_SIZE
    xs = cids[:, 0] + offx
    ys = cids[:, 1] + offy
    pe_rep = jnp.concatenate([pe] * REPL)
    return _pe_lookup(xs, ys, pe_rep)
